# baseline (speedup 1.0000x reference)
import functools

import jax
import jax.numpy as jnp
from jax import lax
from jax.experimental import pallas as pl
from jax.experimental.pallas import tpu as pltpu
from jax.experimental.pallas import tpu_sc as plsc

_B = 4096
_L = 200
_D = 64
_MAX_LEN = 256
_N = _B * _L
_NC = 2
_NS = 16
_NW = _NC * _NS
_ROWS_PER_W = _N // _NW
_SLABS_PER_W = _B // _NW
_G0 = 128
_G1 = _L - _G0


@jax.jit
def _pos_gather(idx_flat, pe):
    mesh = plsc.VectorSubcoreMesh(core_axis_name="c", subcore_axis_name="s")

    nbuf = 4
    lead = 2

    @functools.partial(
        pl.kernel,
        mesh=mesh,
        out_type=jax.ShapeDtypeStruct((_B, _L, _D), jnp.float32),
        compiler_params=pltpu.CompilerParams(use_tc_tiling_on_sc=False),
        scratch_types=[
            pltpu.VMEM((_ROWS_PER_W,), jnp.int32),
            pltpu.VMEM((nbuf, _L, _D), jnp.float32),
            pltpu.VMEM_SHARED((_MAX_LEN, _D), jnp.float32),
            pltpu.SemaphoreType.DMA,
            pltpu.SemaphoreType.DMA,
        ],
    )
    def body(idx_hbm, pe_hbm, out_hbm, idx_v, rows_v, table_sh, gsem, osem):
        wid = lax.axis_index("s") * _NC + lax.axis_index("c")
        bbase = wid * _SLABS_PER_W

        @pl.when(lax.axis_index("s") == 0)
        def _():
            pltpu.sync_copy(pe_hbm, table_sh)

        pltpu.sync_copy(idx_hbm.at[pl.ds(wid * _ROWS_PER_W, _ROWS_PER_W)],
                        idx_v)
        plsc.subcore_barrier()

        def clamp(j):
            def clamp16(i, c):
                o = j * _L + i * 16
                v = idx_v[pl.ds(o, 16)]
                idx_v[pl.ds(o, 16)] = jnp.minimum(v, _MAX_LEN - 1)
                return c

            lax.fori_loop(0, _L // 16, clamp16, 0, unroll=True)

        def start_gather(j):
            clamp(j)
            slot = lax.rem(j, nbuf)
            pltpu.async_copy(table_sh.at[idx_v.at[pl.ds(j * _L, _G0)]],
                             rows_v.at[slot, pl.ds(0, _G0)], gsem)
            pltpu.async_copy(table_sh.at[idx_v.at[pl.ds(j * _L + _G0, _G1)]],
                             rows_v.at[slot, pl.ds(_G0, _G1)], gsem)

        def wait_gather(j):
            slot = lax.rem(j, nbuf)
            pltpu.make_async_copy(table_sh.at[idx_v.at[pl.ds(j * _L, _G0)]],
                                  rows_v.at[slot, pl.ds(0, _G0)], gsem).wait()
            pltpu.make_async_copy(
                table_sh.at[idx_v.at[pl.ds(j * _L + _G0, _G1)]],
                rows_v.at[slot, pl.ds(_G0, _G1)], gsem).wait()

        def start_out(j):
            pltpu.async_copy(rows_v.at[lax.rem(j, nbuf)],
                             out_hbm.at[bbase + j], osem)

        def wait_out(j):
            pltpu.make_async_copy(rows_v.at[lax.rem(j, nbuf)],
                                  out_hbm.at[bbase + j], osem).wait()

        for j in range(lead):
            start_gather(j)

        def step(j, carry):
            @pl.when(j >= nbuf - lead)
            def _():
                wait_out(j - (nbuf - lead))

            @pl.when(j + lead < _SLABS_PER_W)
            def _():
                start_gather(j + lead)

            wait_gather(j)
            start_out(j)
            return carry

        lax.fori_loop(0, _SLABS_PER_W, step, 0)

        for j in range(_SLABS_PER_W - (nbuf - lead), _SLABS_PER_W):
            wait_out(j)

    return body(idx_flat, pe)


def kernel(idx_or_len, pe, device=0):
    idx_flat = idx_or_len.astype(jnp.int32).reshape(_N)
    return _pos_gather(idx_flat, pe.astype(jnp.float32))


# speedup vs baseline: 2.0822x; 2.0822x over previous
import functools

import jax
import jax.numpy as jnp
from jax import lax
from jax.experimental import pallas as pl
from jax.experimental.pallas import tpu as pltpu
from jax.experimental.pallas import tpu_sc as plsc

_B = 4096
_L = 200
_D = 64
_MAX_LEN = 256
_N = _B * _L
_NC = 2
_NS = 16
_NW = _NC * _NS
_ROWS_PER_W = _N // _NW
_SLABS_PER_W = _B // _NW
_G0 = 128
_G1 = _L - _G0


@jax.jit
def _pos_gather(idx_flat, pe):
    mesh = plsc.VectorSubcoreMesh(core_axis_name="c", subcore_axis_name="s")

    nbuf = 4
    lead = 2

    @functools.partial(
        pl.kernel,
        mesh=mesh,
        out_type=jax.ShapeDtypeStruct((_B, _L, 128), jnp.float32),
        compiler_params=pltpu.CompilerParams(use_tc_tiling_on_sc=False),
        scratch_types=[
            pltpu.VMEM((_ROWS_PER_W,), jnp.int32),
            pltpu.VMEM((nbuf, _L, _D), jnp.float32),
            pltpu.VMEM_SHARED((_MAX_LEN, _D), jnp.float32),
            pltpu.SemaphoreType.DMA,
            pltpu.SemaphoreType.DMA,
        ],
    )
    def body(idx_hbm, pe_hbm, out_hbm, idx_v, rows_v, table_sh, gsem, osem):
        wid = lax.axis_index("s") * _NC + lax.axis_index("c")
        bbase = wid * _SLABS_PER_W

        @pl.when(lax.axis_index("s") == 0)
        def _():
            pltpu.sync_copy(pe_hbm, table_sh)

        pltpu.sync_copy(idx_hbm.at[pl.ds(wid * _ROWS_PER_W, _ROWS_PER_W)],
                        idx_v)
        plsc.subcore_barrier()

        def clamp(j):
            def clamp16(i, c):
                o = j * _L + i * 16
                v = idx_v[pl.ds(o, 16)]
                idx_v[pl.ds(o, 16)] = jnp.minimum(v, _MAX_LEN - 1)
                return c

            lax.fori_loop(0, _L // 16, clamp16, 0, unroll=True)

        def start_gather(j):
            clamp(j)
            slot = lax.rem(j, nbuf)
            pltpu.async_copy(table_sh.at[idx_v.at[pl.ds(j * _L, _G0)]],
                             rows_v.at[slot, pl.ds(0, _G0)], gsem)
            pltpu.async_copy(table_sh.at[idx_v.at[pl.ds(j * _L + _G0, _G1)]],
                             rows_v.at[slot, pl.ds(_G0, _G1)], gsem)

        def wait_gather(j):
            slot = lax.rem(j, nbuf)
            pltpu.make_async_copy(table_sh.at[idx_v.at[pl.ds(j * _L, _G0)]],
                                  rows_v.at[slot, pl.ds(0, _G0)], gsem).wait()
            pltpu.make_async_copy(
                table_sh.at[idx_v.at[pl.ds(j * _L + _G0, _G1)]],
                rows_v.at[slot, pl.ds(_G0, _G1)], gsem).wait()

        def start_out(j):
            pltpu.async_copy(rows_v.at[lax.rem(j, nbuf)],
                             out_hbm.at[bbase + j, slice(None), pl.ds(0, _D)],
                             osem)

        def wait_out(j):
            pltpu.make_async_copy(rows_v.at[lax.rem(j, nbuf)],
                                  out_hbm.at[bbase + j, slice(None), pl.ds(0, _D)],
                                  osem).wait()

        for j in range(lead):
            start_gather(j)

        def step(j, carry):
            @pl.when(j >= nbuf - lead)
            def _():
                wait_out(j - (nbuf - lead))

            @pl.when(j + lead < _SLABS_PER_W)
            def _():
                start_gather(j + lead)

            wait_gather(j)
            start_out(j)
            return carry

        lax.fori_loop(0, _SLABS_PER_W, step, 0)

        for j in range(_SLABS_PER_W - (nbuf - lead), _SLABS_PER_W):
            wait_out(j)

    return body(idx_flat, pe)


def kernel(idx_or_len, pe, device=0):
    idx_flat = idx_or_len.astype(jnp.int32).reshape(_N)
    return _pos_gather(idx_flat, pe.astype(jnp.float32))[:, :, :_D]
